# baseline (speedup 1.0000x reference)
import functools

import jax
import jax.numpy as jnp
from jax import lax
from jax.experimental import pallas as pl
from jax.experimental.pallas import tpu as pltpu
from jax.experimental.pallas import tpu_sc as plsc

_EPS = 1e-5
_PREC = lax.Precision.HIGHEST


def _softplus(x):
    return jnp.maximum(x, 0.0) + jnp.log1p(jnp.exp(-jnp.abs(x)))


def _sigmoid(x):
    return jax.nn.sigmoid(x)


_SC_CORES = 2
_SC_SUBCORES = 16


def _sc_gather(x, idx_flat, chunk=200):
    nidx = idx_flat.shape[0]
    d = x.shape[1]
    nw = _SC_CORES * _SC_SUBCORES
    b_per_w = nidx // nw
    nchunk = b_per_w // chunk
    assert b_per_w * nw == nidx and nchunk * chunk == b_per_w and chunk % 8 == 0
    mesh = plsc.VectorSubcoreMesh(core_axis_name="c", subcore_axis_name="s")

    @functools.partial(
        pl.kernel,
        out_type=jax.ShapeDtypeStruct((nidx, d), x.dtype),
        mesh=mesh,
        scratch_types=[
            pltpu.VMEM((chunk,), jnp.int32),
            pltpu.VMEM((chunk, d), jnp.float32),
            pltpu.SemaphoreType.DMA,
        ],
    )
    def gather_kernel(x_hbm, i_hbm, o_hbm, idx_v, rows_v, sem):
        wid = lax.axis_index("s") * _SC_CORES + lax.axis_index("c")
        base = wid * b_per_w

        @pl.loop(0, nchunk)
        def _(k):
            off = pl.multiple_of(base + k * chunk, 8)
            pltpu.sync_copy(i_hbm.at[pl.ds(off, chunk)], idx_v)
            pltpu.async_copy(x_hbm.at[idx_v], rows_v, sem).wait()
            pltpu.sync_copy(rows_v, o_hbm.at[pl.ds(off, chunk)])

    return gather_kernel(x, idx_flat)


def _embed_proj(atom, w_emb, b_emb, wn, b1, blk=2000):
    n = atom.shape[0]
    d = w_emb.shape[1]
    d2 = wn.shape[1]

    def body(a_ref, we_ref, be_ref, wn_ref, b1_ref, ox_ref, oy_ref):
        x = (
            jnp.dot(a_ref[...], we_ref[...], preferred_element_type=jnp.float32,
                    precision=_PREC)
            + be_ref[...]
        )
        ox_ref[...] = x
        oy_ref[...] = (
            jnp.dot(x, wn_ref[...], preferred_element_type=jnp.float32,
                    precision=_PREC)
            + b1_ref[...]
        )

    return pl.pallas_call(
        body,
        grid=(n // blk,),
        in_specs=[
            pl.BlockSpec((blk, atom.shape[1]), lambda i: (i, 0)),
            pl.BlockSpec(w_emb.shape, lambda i: (0, 0)),
            pl.BlockSpec(b_emb.shape, lambda i: (0, 0)),
            pl.BlockSpec(wn.shape, lambda i: (0, 0)),
            pl.BlockSpec(b1.shape, lambda i: (0, 0)),
        ],
        out_specs=[
            pl.BlockSpec((blk, d), lambda i: (i, 0)),
            pl.BlockSpec((blk, d2), lambda i: (i, 0)),
        ],
        out_shape=[
            jax.ShapeDtypeStruct((n, d), jnp.float32),
            jax.ShapeDtypeStruct((n, d2), jnp.float32),
        ],
    )(atom, w_emb, b_emb, wn, b1)


def _conv_stats(x, g3, e3, ws, we, blk):
    n, d = x.shape
    m, _, d2 = g3.shape
    nb = n // blk

    def body(x_ref, g_ref, e_ref, ws_ref, we_ref, os_ref, oq_ref):
        ib = pl.program_id(0)
        s = jnp.dot(x_ref[...], ws_ref[...], preferred_element_type=jnp.float32,
                    precision=_PREC)
        acc_s = jnp.zeros((1, d2), jnp.float32)
        acc_q = jnp.zeros((1, d2), jnp.float32)
        for j in range(m):
            gated = s + g_ref[j] + jnp.dot(
                e_ref[j], we_ref[...], preferred_element_type=jnp.float32,
                precision=_PREC)
            acc_s += jnp.sum(gated, axis=0, keepdims=True)
            acc_q += jnp.sum(gated * gated, axis=0, keepdims=True)
        first = ib == 0
        os_ref[...] = jnp.where(first, 0.0, os_ref[...]) + acc_s
        oq_ref[...] = jnp.where(first, 0.0, oq_ref[...]) + acc_q

    return pl.pallas_call(
        body,
        grid=(nb,),
        in_specs=[
            pl.BlockSpec((blk, d), lambda ib: (ib, 0)),
            pl.BlockSpec((m, blk, d2), lambda ib: (0, ib, 0)),
            pl.BlockSpec((m, blk, e3.shape[2]), lambda ib: (0, ib, 0)),
            pl.BlockSpec(ws.shape, lambda ib: (0, 0)),
            pl.BlockSpec(we.shape, lambda ib: (0, 0)),
        ],
        out_specs=[
            pl.BlockSpec((1, d2), lambda ib: (0, 0)),
            pl.BlockSpec((1, d2), lambda ib: (0, 0)),
        ],
        out_shape=[jax.ShapeDtypeStruct((1, d2), jnp.float32)] * 2,
    )(x, g3, e3, ws, we)


def _conv_apply(x, g3, e3, ws, we, s1, q1, g1, b1p, blk):
    n, d = x.shape
    m, _, d2 = g3.shape
    nb = n // blk
    cnt = float(n * m)

    def body(
        x_ref, g_ref, e_ref, ws_ref, we_ref,
        s1_ref, q1_ref, g1_ref, b1p_ref,
        ons_ref, os_ref, oq_ref,
    ):
        ib = pl.program_id(0)
        mu = s1_ref[...] / cnt
        var = q1_ref[...] / cnt - mu * mu
        scale = g1_ref[...] * lax.rsqrt(var + _EPS)
        shift = b1p_ref[...] - mu * scale
        s = jnp.dot(x_ref[...], ws_ref[...], preferred_element_type=jnp.float32,
                    precision=_PREC)
        acc = jnp.zeros((x_ref.shape[0], d), jnp.float32)
        for j in range(m):
            gated = s + g_ref[j] + jnp.dot(
                e_ref[j], we_ref[...], preferred_element_type=jnp.float32,
                precision=_PREC)
            gn = gated * scale + shift
            acc += _sigmoid(gn[:, :d]) * _softplus(gn[:, d:])
        ons_ref[...] = acc
        cs = jnp.sum(acc, axis=0, keepdims=True)
        cq = jnp.sum(acc * acc, axis=0, keepdims=True)
        first = ib == 0
        os_ref[...] = jnp.where(first, 0.0, os_ref[...]) + cs
        oq_ref[...] = jnp.where(first, 0.0, oq_ref[...]) + cq

    return pl.pallas_call(
        body,
        grid=(nb,),
        in_specs=[
            pl.BlockSpec((blk, d), lambda ib: (ib, 0)),
            pl.BlockSpec((m, blk, d2), lambda ib: (0, ib, 0)),
            pl.BlockSpec((m, blk, e3.shape[2]), lambda ib: (0, ib, 0)),
            pl.BlockSpec(ws.shape, lambda ib: (0, 0)),
            pl.BlockSpec(we.shape, lambda ib: (0, 0)),
            pl.BlockSpec((1, d2), lambda ib: (0, 0)),
            pl.BlockSpec((1, d2), lambda ib: (0, 0)),
            pl.BlockSpec((1, d2), lambda ib: (0, 0)),
            pl.BlockSpec((1, d2), lambda ib: (0, 0)),
        ],
        out_specs=[
            pl.BlockSpec((blk, d), lambda ib: (ib, 0)),
            pl.BlockSpec((1, d), lambda ib: (0, 0)),
            pl.BlockSpec((1, d), lambda ib: (0, 0)),
        ],
        out_shape=[
            jax.ShapeDtypeStruct((n, d), jnp.float32),
            jax.ShapeDtypeStruct((1, d), jnp.float32),
            jax.ShapeDtypeStruct((1, d), jnp.float32),
        ],
    )(x, g3, e3, ws, we, s1, q1, g1, b1p)


def _conv_resid(x, ns, s2, q2, g2, b2, wn, b1, blk=2000):
    n, d = x.shape
    nb = n // blk
    cnt = float(n)
    with_proj = wn is not None

    def body(x_ref, ns_ref, s2_ref, q2_ref, g2_ref, b2_ref, *rest):
        mu = s2_ref[...] / cnt
        var = q2_ref[...] / cnt - mu * mu
        scale = g2_ref[...] * lax.rsqrt(var + _EPS)
        shift = b2_ref[...] - mu * scale
        xn = _softplus(x_ref[...] + ns_ref[...] * scale + shift)
        if with_proj:
            wn_ref, b1_ref, ox_ref, oy_ref = rest
            ox_ref[...] = xn
            oy_ref[...] = (
                jnp.dot(xn, wn_ref[...], preferred_element_type=jnp.float32,
                        precision=_PREC)
                + b1_ref[...]
            )
        else:
            (ox_ref,) = rest
            ox_ref[...] = xn

    in_specs = [
        pl.BlockSpec((blk, d), lambda i: (i, 0)),
        pl.BlockSpec((blk, d), lambda i: (i, 0)),
        pl.BlockSpec((1, d), lambda i: (0, 0)),
        pl.BlockSpec((1, d), lambda i: (0, 0)),
        pl.BlockSpec((1, d), lambda i: (0, 0)),
        pl.BlockSpec((1, d), lambda i: (0, 0)),
    ]
    args = [x, ns, s2, q2, g2, b2]
    if with_proj:
        d2 = wn.shape[1]
        in_specs += [
            pl.BlockSpec(wn.shape, lambda i: (0, 0)),
            pl.BlockSpec(b1.shape, lambda i: (0, 0)),
        ]
        args += [wn, b1]
        out_specs = [
            pl.BlockSpec((blk, d), lambda i: (i, 0)),
            pl.BlockSpec((blk, d2), lambda i: (i, 0)),
        ]
        out_shape = [
            jax.ShapeDtypeStruct((n, d), jnp.float32),
            jax.ShapeDtypeStruct((n, d2), jnp.float32),
        ]
    else:
        out_specs = pl.BlockSpec((blk, d), lambda i: (i, 0))
        out_shape = jax.ShapeDtypeStruct((n, d), jnp.float32)

    return pl.pallas_call(
        body,
        grid=(nb,),
        in_specs=in_specs,
        out_specs=out_specs,
        out_shape=out_shape,
    )(*args)


def _tail(x3, cfg_in, wfc, bfc, wcfg, bcfg, wfus, bfus, wout_t, bout):
    bsz, per, d = x3.shape

    def body(x3_ref, c_ref, wfc_ref, bfc_ref, wcfg_ref, bcfg_ref,
             wfus_ref, bfus_ref, wout_ref, bout_ref, o_ref):
        pooled = jnp.mean(x3_ref[...], axis=1)
        emb = _softplus(
            jnp.dot(pooled, wfc_ref[...], preferred_element_type=jnp.float32,
                    precision=_PREC)
            + bfc_ref[...]
        )
        cfg = jnp.maximum(
            jnp.dot(c_ref[...], wcfg_ref[...], preferred_element_type=jnp.float32,
                    precision=_PREC)
            + bcfg_ref[...],
            0.0,
        )
        fused = jnp.concatenate([emb, cfg], axis=1)
        fused = jnp.maximum(
            jnp.dot(fused, wfus_ref[...], preferred_element_type=jnp.float32,
                    precision=_PREC)
            + bfus_ref[...],
            0.0,
        )
        out = jnp.sum(fused * wout_ref[...], axis=1, keepdims=True) + bout_ref[...]
        o_ref[...] = out

    return pl.pallas_call(
        body,
        in_specs=[
            pl.BlockSpec(x3.shape, lambda: (0, 0, 0)),
            pl.BlockSpec(cfg_in.shape, lambda: (0, 0)),
            pl.BlockSpec(wfc.shape, lambda: (0, 0)),
            pl.BlockSpec(bfc.shape, lambda: (0, 0)),
            pl.BlockSpec(wcfg.shape, lambda: (0, 0)),
            pl.BlockSpec(bcfg.shape, lambda: (0, 0)),
            pl.BlockSpec(wfus.shape, lambda: (0, 0)),
            pl.BlockSpec(bfus.shape, lambda: (0, 0)),
            pl.BlockSpec(wout_t.shape, lambda: (0, 0)),
            pl.BlockSpec(bout.shape, lambda: (0, 0)),
        ],
        out_specs=pl.BlockSpec((bsz, 1), lambda: (0, 0)),
        out_shape=jax.ShapeDtypeStruct((bsz, 1), jnp.float32),
    )(x3, cfg_in, wfc, bfc, wcfg, bcfg, wfus, bfus, wout_t, bout)


def kernel(atom, nbr, idx, crys_idx, config_vector, mono_bg, W_emb, b_emb,
           conv_W, conv_b, bn1_g, bn1_b, bn2_g, bn2_b, W_fc, b_fc, W_cfg,
           b_cfg, W_prop, b_prop, W_fus, b_fus, W_out, b_out):
    n, _ = atom.shape
    m = idx.shape[1]
    d = W_emb.shape[1]
    nbrf = nbr.shape[2]
    nconv = conv_W.shape[0]
    blk = 1000

    wns = [conv_W[l][d:2 * d] for l in range(nconv)]
    wss = [conv_W[l][:d] for l in range(nconv)]
    wes = [conv_W[l][2 * d:] for l in range(nconv)]
    b1s = [conv_b[l].reshape(1, -1) for l in range(nconv)]

    x, y = _embed_proj(atom, W_emb, b_emb.reshape(1, -1), wns[0], b1s[0])

    idx_t = idx.T.reshape(-1)
    nbr_t = jnp.transpose(nbr, (1, 0, 2))

    for l in range(nconv):
        g3 = _sc_gather(y, idx_t).reshape(m, n, 2 * d)
        s1, q1 = _conv_stats(x, g3, nbr_t, wss[l], wes[l], blk)
        ns, s2, q2 = _conv_apply(
            x, g3, nbr_t, wss[l], wes[l], s1, q1,
            bn1_g[l].reshape(1, -1), bn1_b[l].reshape(1, -1), blk,
        )
        last = l == nconv - 1
        res = _conv_resid(
            x, ns, s2, q2, bn2_g[l].reshape(1, -1), bn2_b[l].reshape(1, -1),
            None if last else wns[l + 1], None if last else b1s[l + 1],
        )
        x = res if last else res[0]
        y = None if last else res[1]

    bsz = crys_idx.shape[0]
    x3 = x.reshape(bsz, n // bsz, d)
    return _tail(
        x3, config_vector, W_fc, b_fc.reshape(1, -1), W_cfg,
        b_cfg.reshape(1, -1), W_fus, b_fus.reshape(1, -1),
        W_out.T, b_out.reshape(1, 1),
    )


# speedup vs baseline: 1.3565x; 1.3565x over previous
import functools

import jax
import jax.numpy as jnp
from jax import lax
from jax.experimental import pallas as pl
from jax.experimental.pallas import tpu as pltpu
from jax.experimental.pallas import tpu_sc as plsc

_EPS = 1e-5
_PREC = lax.Precision.HIGHEST


def _softplus(x):
    return jnp.maximum(x, 0.0) + jnp.log1p(jnp.exp(-jnp.abs(x)))


def _sigmoid(x):
    return jax.nn.sigmoid(x)


_SC_CORES = 2
_SC_SUBCORES = 16


def _sc_gather(x, idx_flat, chunk=200):
    nidx = idx_flat.shape[0]
    d = x.shape[1]
    nw = _SC_CORES * _SC_SUBCORES
    b_per_w = nidx // nw
    nchunk = b_per_w // chunk
    assert b_per_w * nw == nidx and nchunk * chunk == b_per_w and chunk % 8 == 0
    mesh = plsc.VectorSubcoreMesh(core_axis_name="c", subcore_axis_name="s")

    @functools.partial(
        pl.kernel,
        out_type=jax.ShapeDtypeStruct((nidx, d), x.dtype),
        mesh=mesh,
        scratch_types=[
            pltpu.VMEM((chunk,), jnp.int32),
            pltpu.VMEM((chunk, d), jnp.float32),
            pltpu.SemaphoreType.DMA,
        ],
    )
    def gather_kernel(x_hbm, i_hbm, o_hbm, idx_v, rows_v, sem):
        wid = lax.axis_index("s") * _SC_CORES + lax.axis_index("c")
        base = wid * b_per_w

        @pl.loop(0, nchunk)
        def _(k):
            off = pl.multiple_of(base + k * chunk, 8)
            pltpu.sync_copy(i_hbm.at[pl.ds(off, chunk)], idx_v)
            pltpu.async_copy(x_hbm.at[idx_v], rows_v, sem).wait()
            pltpu.sync_copy(rows_v, o_hbm.at[pl.ds(off, chunk)])

    return gather_kernel(x, idx_flat)


def _embed_proj(atom, w_emb, b_emb, wn, b1, blk=2000):
    n = atom.shape[0]
    d = w_emb.shape[1]
    d2 = wn.shape[1]

    def body(a_ref, we_ref, be_ref, wn_ref, b1_ref, ox_ref, oy_ref):
        x = (
            jnp.dot(a_ref[...], we_ref[...], preferred_element_type=jnp.float32,
                    precision=_PREC)
            + be_ref[...]
        )
        ox_ref[...] = x
        oy_ref[...] = (
            jnp.dot(x, wn_ref[...], preferred_element_type=jnp.float32,
                    precision=_PREC)
            + b1_ref[...]
        )

    return pl.pallas_call(
        body,
        grid=(n // blk,),
        in_specs=[
            pl.BlockSpec((blk, atom.shape[1]), lambda i: (i, 0)),
            pl.BlockSpec(w_emb.shape, lambda i: (0, 0)),
            pl.BlockSpec(b_emb.shape, lambda i: (0, 0)),
            pl.BlockSpec(wn.shape, lambda i: (0, 0)),
            pl.BlockSpec(b1.shape, lambda i: (0, 0)),
        ],
        out_specs=[
            pl.BlockSpec((blk, d), lambda i: (i, 0)),
            pl.BlockSpec((blk, d2), lambda i: (i, 0)),
        ],
        out_shape=[
            jax.ShapeDtypeStruct((n, d), jnp.float32),
            jax.ShapeDtypeStruct((n, d2), jnp.float32),
        ],
    )(atom, w_emb, b_emb, wn, b1)


def _conv_stats(x, g, e, ws, we, blk):
    n, d = x.shape
    d2 = g.shape[1]
    m = g.shape[0] // n
    nb = n // blk

    def body(x_ref, g_ref, e_ref, ws_ref, we_ref, os_ref, oq_ref, s_ref):
        ib = pl.program_id(0)
        j = pl.program_id(1)

        @pl.when(j == 0)
        def _():
            s_ref[...] = jnp.dot(
                x_ref[...], ws_ref[...], preferred_element_type=jnp.float32,
                precision=_PREC)

        gated = s_ref[...] + g_ref[...] + jnp.dot(
            e_ref[...], we_ref[...], preferred_element_type=jnp.float32)
        cs = jnp.sum(gated, axis=0, keepdims=True)
        cq = jnp.sum(gated * gated, axis=0, keepdims=True)
        first = jnp.logical_and(ib == 0, j == 0)
        os_ref[...] = jnp.where(first, 0.0, os_ref[...]) + cs
        oq_ref[...] = jnp.where(first, 0.0, oq_ref[...]) + cq

    return pl.pallas_call(
        body,
        grid=(nb, m),
        in_specs=[
            pl.BlockSpec((blk, d), lambda ib, j: (ib, 0)),
            pl.BlockSpec((blk, d2), lambda ib, j: (j * nb + ib, 0)),
            pl.BlockSpec((blk, e.shape[1]), lambda ib, j: (j * nb + ib, 0)),
            pl.BlockSpec(ws.shape, lambda ib, j: (0, 0)),
            pl.BlockSpec(we.shape, lambda ib, j: (0, 0)),
        ],
        out_specs=[
            pl.BlockSpec((1, d2), lambda ib, j: (0, 0)),
            pl.BlockSpec((1, d2), lambda ib, j: (0, 0)),
        ],
        out_shape=[jax.ShapeDtypeStruct((1, d2), jnp.float32)] * 2,
        scratch_shapes=[pltpu.VMEM((blk, d2), jnp.float32)],
    )(x, g, e, ws, we)


def _conv_apply(x, g, e, ws, we, s1, q1, g1, b1p, blk):
    n, d = x.shape
    d2 = g.shape[1]
    m = g.shape[0] // n
    nb = n // blk
    cnt = float(n * m)

    def body(
        x_ref, g_ref, e_ref, ws_ref, we_ref,
        s1_ref, q1_ref, g1_ref, b1p_ref,
        ons_ref, os_ref, oq_ref, s_ref,
    ):
        ib = pl.program_id(0)
        j = pl.program_id(1)

        @pl.when(j == 0)
        def _():
            s_ref[...] = jnp.dot(
                x_ref[...], ws_ref[...], preferred_element_type=jnp.float32,
                precision=_PREC)

        mu = s1_ref[...] / cnt
        var = q1_ref[...] / cnt - mu * mu
        scale = g1_ref[...] * lax.rsqrt(var + _EPS)
        shift = b1p_ref[...] - mu * scale

        gated = s_ref[...] + g_ref[...] + jnp.dot(
            e_ref[...], we_ref[...], preferred_element_type=jnp.float32)
        gn = gated * scale + shift
        contrib = _sigmoid(gn[:, :d]) * _softplus(gn[:, d:])
        cur = jnp.where(j == 0, 0.0, ons_ref[...]) + contrib
        ons_ref[...] = cur

        @pl.when(j == m - 1)
        def _():
            cs = jnp.sum(cur, axis=0, keepdims=True)
            cq = jnp.sum(cur * cur, axis=0, keepdims=True)
            first = ib == 0
            os_ref[...] = jnp.where(first, 0.0, os_ref[...]) + cs
            oq_ref[...] = jnp.where(first, 0.0, oq_ref[...]) + cq

    return pl.pallas_call(
        body,
        grid=(nb, m),
        in_specs=[
            pl.BlockSpec((blk, d), lambda ib, j: (ib, 0)),
            pl.BlockSpec((blk, d2), lambda ib, j: (j * nb + ib, 0)),
            pl.BlockSpec((blk, e.shape[1]), lambda ib, j: (j * nb + ib, 0)),
            pl.BlockSpec(ws.shape, lambda ib, j: (0, 0)),
            pl.BlockSpec(we.shape, lambda ib, j: (0, 0)),
            pl.BlockSpec((1, d2), lambda ib, j: (0, 0)),
            pl.BlockSpec((1, d2), lambda ib, j: (0, 0)),
            pl.BlockSpec((1, d2), lambda ib, j: (0, 0)),
            pl.BlockSpec((1, d2), lambda ib, j: (0, 0)),
        ],
        out_specs=[
            pl.BlockSpec((blk, d), lambda ib, j: (ib, 0)),
            pl.BlockSpec((1, d), lambda ib, j: (0, 0)),
            pl.BlockSpec((1, d), lambda ib, j: (0, 0)),
        ],
        out_shape=[
            jax.ShapeDtypeStruct((n, d), jnp.float32),
            jax.ShapeDtypeStruct((1, d), jnp.float32),
            jax.ShapeDtypeStruct((1, d), jnp.float32),
        ],
        scratch_shapes=[pltpu.VMEM((blk, d2), jnp.float32)],
    )(x, g, e, ws, we, s1, q1, g1, b1p)


def _conv_resid(x, ns, s2, q2, g2, b2, wn, b1, blk=2000):
    n, d = x.shape
    nb = n // blk
    cnt = float(n)
    with_proj = wn is not None

    def body(x_ref, ns_ref, s2_ref, q2_ref, g2_ref, b2_ref, *rest):
        mu = s2_ref[...] / cnt
        var = q2_ref[...] / cnt - mu * mu
        scale = g2_ref[...] * lax.rsqrt(var + _EPS)
        shift = b2_ref[...] - mu * scale
        xn = _softplus(x_ref[...] + ns_ref[...] * scale + shift)
        if with_proj:
            wn_ref, b1_ref, ox_ref, oy_ref = rest
            ox_ref[...] = xn
            oy_ref[...] = (
                jnp.dot(xn, wn_ref[...], preferred_element_type=jnp.float32,
                        precision=_PREC)
                + b1_ref[...]
            )
        else:
            (ox_ref,) = rest
            ox_ref[...] = xn

    in_specs = [
        pl.BlockSpec((blk, d), lambda i: (i, 0)),
        pl.BlockSpec((blk, d), lambda i: (i, 0)),
        pl.BlockSpec((1, d), lambda i: (0, 0)),
        pl.BlockSpec((1, d), lambda i: (0, 0)),
        pl.BlockSpec((1, d), lambda i: (0, 0)),
        pl.BlockSpec((1, d), lambda i: (0, 0)),
    ]
    args = [x, ns, s2, q2, g2, b2]
    if with_proj:
        d2 = wn.shape[1]
        in_specs += [
            pl.BlockSpec(wn.shape, lambda i: (0, 0)),
            pl.BlockSpec(b1.shape, lambda i: (0, 0)),
        ]
        args += [wn, b1]
        out_specs = [
            pl.BlockSpec((blk, d), lambda i: (i, 0)),
            pl.BlockSpec((blk, d2), lambda i: (i, 0)),
        ]
        out_shape = [
            jax.ShapeDtypeStruct((n, d), jnp.float32),
            jax.ShapeDtypeStruct((n, d2), jnp.float32),
        ]
    else:
        out_specs = pl.BlockSpec((blk, d), lambda i: (i, 0))
        out_shape = jax.ShapeDtypeStruct((n, d), jnp.float32)

    return pl.pallas_call(
        body,
        grid=(nb,),
        in_specs=in_specs,
        out_specs=out_specs,
        out_shape=out_shape,
    )(*args)


def _tail(x3, cfg_in, wfc, bfc, wcfg, bcfg, wfus, bfus, wout_t, bout):
    bsz, per, d = x3.shape

    def body(x3_ref, c_ref, wfc_ref, bfc_ref, wcfg_ref, bcfg_ref,
             wfus_ref, bfus_ref, wout_ref, bout_ref, o_ref):
        pooled = jnp.mean(x3_ref[...], axis=1)
        emb = _softplus(
            jnp.dot(pooled, wfc_ref[...], preferred_element_type=jnp.float32,
                    precision=_PREC)
            + bfc_ref[...]
        )
        cfg = jnp.maximum(
            jnp.dot(c_ref[...], wcfg_ref[...], preferred_element_type=jnp.float32,
                    precision=_PREC)
            + bcfg_ref[...],
            0.0,
        )
        fused = jnp.concatenate([emb, cfg], axis=1)
        fused = jnp.maximum(
            jnp.dot(fused, wfus_ref[...], preferred_element_type=jnp.float32,
                    precision=_PREC)
            + bfus_ref[...],
            0.0,
        )
        out = jnp.sum(fused * wout_ref[...], axis=1, keepdims=True) + bout_ref[...]
        o_ref[...] = out

    return pl.pallas_call(
        body,
        in_specs=[
            pl.BlockSpec(x3.shape, lambda: (0, 0, 0)),
            pl.BlockSpec(cfg_in.shape, lambda: (0, 0)),
            pl.BlockSpec(wfc.shape, lambda: (0, 0)),
            pl.BlockSpec(bfc.shape, lambda: (0, 0)),
            pl.BlockSpec(wcfg.shape, lambda: (0, 0)),
            pl.BlockSpec(bcfg.shape, lambda: (0, 0)),
            pl.BlockSpec(wfus.shape, lambda: (0, 0)),
            pl.BlockSpec(bfus.shape, lambda: (0, 0)),
            pl.BlockSpec(wout_t.shape, lambda: (0, 0)),
            pl.BlockSpec(bout.shape, lambda: (0, 0)),
        ],
        out_specs=pl.BlockSpec((bsz, 1), lambda: (0, 0)),
        out_shape=jax.ShapeDtypeStruct((bsz, 1), jnp.float32),
    )(x3, cfg_in, wfc, bfc, wcfg, bcfg, wfus, bfus, wout_t, bout)


def kernel(atom, nbr, idx, crys_idx, config_vector, mono_bg, W_emb, b_emb,
           conv_W, conv_b, bn1_g, bn1_b, bn2_g, bn2_b, W_fc, b_fc, W_cfg,
           b_cfg, W_prop, b_prop, W_fus, b_fus, W_out, b_out):
    n, _ = atom.shape
    m = idx.shape[1]
    d = W_emb.shape[1]
    nbrf = nbr.shape[2]
    nconv = conv_W.shape[0]
    blk = 2000

    wns = [conv_W[l][d:2 * d] for l in range(nconv)]
    wss = [conv_W[l][:d] for l in range(nconv)]
    wes = [conv_W[l][2 * d:] for l in range(nconv)]
    b1s = [conv_b[l].reshape(1, -1) for l in range(nconv)]

    x, y = _embed_proj(atom, W_emb, b_emb.reshape(1, -1), wns[0], b1s[0])

    idx_t = idx.T.reshape(-1)
    nbr_t = jnp.transpose(nbr, (1, 0, 2)).reshape(m * n, nbrf)

    for l in range(nconv):
        g = _sc_gather(y, idx_t)
        s1, q1 = _conv_stats(x, g, nbr_t, wss[l], wes[l], blk)
        ns, s2, q2 = _conv_apply(
            x, g, nbr_t, wss[l], wes[l], s1, q1,
            bn1_g[l].reshape(1, -1), bn1_b[l].reshape(1, -1), blk,
        )
        last = l == nconv - 1
        res = _conv_resid(
            x, ns, s2, q2, bn2_g[l].reshape(1, -1), bn2_b[l].reshape(1, -1),
            None if last else wns[l + 1], None if last else b1s[l + 1],
        )
        x = res if last else res[0]
        y = None if last else res[1]

    bsz = crys_idx.shape[0]
    x3 = x.reshape(bsz, n // bsz, d)
    return _tail(
        x3, config_vector, W_fc, b_fc.reshape(1, -1), W_cfg,
        b_cfg.reshape(1, -1), W_fus, b_fus.reshape(1, -1),
        W_out.T, b_out.reshape(1, 1),
    )


# speedup vs baseline: 1.6807x; 1.2390x over previous
import functools

import jax
import jax.numpy as jnp
from jax import lax
from jax.experimental import pallas as pl
from jax.experimental.pallas import tpu as pltpu
from jax.experimental.pallas import tpu_sc as plsc

_EPS = 1e-5
_PREC = lax.Precision.HIGHEST


def _softplus(x):
    return jnp.maximum(x, 0.0) + jnp.log1p(jnp.exp(-jnp.abs(x)))


def _sigmoid(x):
    return jax.nn.sigmoid(x)


_SC_CORES = 2
_SC_SUBCORES = 16


def _sc_gather(x, idx_flat, chunk=400):
    nidx = idx_flat.shape[0]
    d = x.shape[1]
    nw = _SC_CORES * _SC_SUBCORES
    nc = nidx // chunk
    assert nc * chunk == nidx and chunk % 8 == 0
    npair = -(-nc // (2 * nw))
    mesh = plsc.VectorSubcoreMesh(core_axis_name="c", subcore_axis_name="s")

    @functools.partial(
        pl.kernel,
        out_type=jax.ShapeDtypeStruct((nidx, d), x.dtype),
        mesh=mesh,
        scratch_types=[
            pltpu.VMEM((chunk,), jnp.int32),
            pltpu.VMEM((chunk,), jnp.int32),
            pltpu.VMEM((chunk, d), x.dtype),
            pltpu.VMEM((chunk, d), x.dtype),
            pltpu.SemaphoreType.DMA,
            pltpu.SemaphoreType.DMA,
            pltpu.SemaphoreType.DMA,
            pltpu.SemaphoreType.DMA,
        ],
    )
    def gather_kernel(x_hbm, i_hbm, o_hbm, idx_a, idx_b, rows_a, rows_b,
                      gs_a, gs_b, ss_a, ss_b):
        wid = lax.axis_index("s") * _SC_CORES + lax.axis_index("c")

        @pl.loop(0, npair)
        def _(p):
            c_a = wid + (2 * p) * nw
            c_b = wid + (2 * p + 1) * nw
            off_a = pl.multiple_of(c_a * chunk, 8)
            off_b = pl.multiple_of(c_b * chunk, 8)

            @pl.when(c_a < nc)
            def _():
                pltpu.sync_copy(i_hbm.at[pl.ds(off_a, chunk)], idx_a)
                pltpu.make_async_copy(x_hbm.at[idx_a], rows_a, gs_a).start()

            @pl.when(c_b < nc)
            def _():
                pltpu.sync_copy(i_hbm.at[pl.ds(off_b, chunk)], idx_b)
                pltpu.make_async_copy(x_hbm.at[idx_b], rows_b, gs_b).start()

            @pl.when(c_a < nc)
            def _():
                pltpu.make_async_copy(x_hbm.at[idx_a], rows_a, gs_a).wait()
                pltpu.make_async_copy(rows_a, o_hbm.at[pl.ds(off_a, chunk)], ss_a).start()

            @pl.when(c_b < nc)
            def _():
                pltpu.make_async_copy(x_hbm.at[idx_b], rows_b, gs_b).wait()
                pltpu.make_async_copy(rows_b, o_hbm.at[pl.ds(off_b, chunk)], ss_b).start()

            @pl.when(c_a < nc)
            def _():
                pltpu.make_async_copy(rows_a, o_hbm.at[pl.ds(off_a, chunk)], ss_a).wait()

            @pl.when(c_b < nc)
            def _():
                pltpu.make_async_copy(rows_b, o_hbm.at[pl.ds(off_b, chunk)], ss_b).wait()

    return gather_kernel(x, idx_flat)


def _embed_proj(atom, w_emb, b_emb, wn, b1, blk=2000):
    n = atom.shape[0]
    d = w_emb.shape[1]
    d2 = wn.shape[1]

    def body(a_ref, we_ref, be_ref, wn_ref, b1_ref, ox_ref, oy_ref):
        x = (
            jnp.dot(a_ref[...], we_ref[...], preferred_element_type=jnp.float32,
                    precision=_PREC)
            + be_ref[...]
        )
        ox_ref[...] = x
        oy_ref[...] = (
            jnp.dot(x, wn_ref[...], preferred_element_type=jnp.float32,
                    precision=_PREC)
            + b1_ref[...]
        )

    return pl.pallas_call(
        body,
        grid=(n // blk,),
        in_specs=[
            pl.BlockSpec((blk, atom.shape[1]), lambda i: (i, 0)),
            pl.BlockSpec(w_emb.shape, lambda i: (0, 0)),
            pl.BlockSpec(b_emb.shape, lambda i: (0, 0)),
            pl.BlockSpec(wn.shape, lambda i: (0, 0)),
            pl.BlockSpec(b1.shape, lambda i: (0, 0)),
        ],
        out_specs=[
            pl.BlockSpec((blk, d), lambda i: (i, 0)),
            pl.BlockSpec((blk, d2), lambda i: (i, 0)),
        ],
        out_shape=[
            jax.ShapeDtypeStruct((n, d), jnp.float32),
            jax.ShapeDtypeStruct((n, d2), jnp.float32),
        ],
    )(atom, w_emb, b_emb, wn, b1)


def _conv_stats(x, g, e, ws, we, blk):
    n, d = x.shape
    d2 = g.shape[1]
    m = g.shape[0] // n
    nb = n // blk

    def body(x_ref, g_ref, e_ref, ws_ref, we_ref, os_ref, oq_ref, s_ref):
        ib = pl.program_id(0)
        j = pl.program_id(1)

        @pl.when(j == 0)
        def _():
            s_ref[...] = jnp.dot(
                x_ref[...], ws_ref[...], preferred_element_type=jnp.float32,
                precision=_PREC)

        gated = s_ref[...] + g_ref[...] + jnp.dot(
            e_ref[...], we_ref[...], preferred_element_type=jnp.float32)
        cs = jnp.sum(gated, axis=0, keepdims=True)
        cq = jnp.sum(gated * gated, axis=0, keepdims=True)
        first = jnp.logical_and(ib == 0, j == 0)
        os_ref[...] = jnp.where(first, 0.0, os_ref[...]) + cs
        oq_ref[...] = jnp.where(first, 0.0, oq_ref[...]) + cq

    return pl.pallas_call(
        body,
        grid=(nb, m),
        in_specs=[
            pl.BlockSpec((blk, d), lambda ib, j: (ib, 0)),
            pl.BlockSpec((blk, d2), lambda ib, j: (j * nb + ib, 0)),
            pl.BlockSpec((blk, e.shape[1]), lambda ib, j: (j * nb + ib, 0)),
            pl.BlockSpec(ws.shape, lambda ib, j: (0, 0)),
            pl.BlockSpec(we.shape, lambda ib, j: (0, 0)),
        ],
        out_specs=[
            pl.BlockSpec((1, d2), lambda ib, j: (0, 0)),
            pl.BlockSpec((1, d2), lambda ib, j: (0, 0)),
        ],
        out_shape=[jax.ShapeDtypeStruct((1, d2), jnp.float32)] * 2,
        scratch_shapes=[pltpu.VMEM((blk, d2), jnp.float32)],
    )(x, g, e, ws, we)


def _conv_apply(x, g, e, ws, we, s1, q1, g1, b1p, blk):
    n, d = x.shape
    d2 = g.shape[1]
    m = g.shape[0] // n
    nb = n // blk
    cnt = float(n * m)

    def body(
        x_ref, g_ref, e_ref, ws_ref, we_ref,
        s1_ref, q1_ref, g1_ref, b1p_ref,
        ons_ref, os_ref, oq_ref, s_ref,
    ):
        ib = pl.program_id(0)
        j = pl.program_id(1)

        @pl.when(j == 0)
        def _():
            s_ref[...] = jnp.dot(
                x_ref[...], ws_ref[...], preferred_element_type=jnp.float32,
                precision=_PREC)

        mu = s1_ref[...] / cnt
        var = q1_ref[...] / cnt - mu * mu
        scale = g1_ref[...] * lax.rsqrt(var + _EPS)
        shift = b1p_ref[...] - mu * scale

        gated = s_ref[...] + g_ref[...] + jnp.dot(
            e_ref[...], we_ref[...], preferred_element_type=jnp.float32)
        gn = gated * scale + shift
        contrib = _sigmoid(gn[:, :d]) * _softplus(gn[:, d:])
        cur = jnp.where(j == 0, 0.0, ons_ref[...]) + contrib
        ons_ref[...] = cur

        @pl.when(j == m - 1)
        def _():
            cs = jnp.sum(cur, axis=0, keepdims=True)
            cq = jnp.sum(cur * cur, axis=0, keepdims=True)
            first = ib == 0
            os_ref[...] = jnp.where(first, 0.0, os_ref[...]) + cs
            oq_ref[...] = jnp.where(first, 0.0, oq_ref[...]) + cq

    return pl.pallas_call(
        body,
        grid=(nb, m),
        in_specs=[
            pl.BlockSpec((blk, d), lambda ib, j: (ib, 0)),
            pl.BlockSpec((blk, d2), lambda ib, j: (j * nb + ib, 0)),
            pl.BlockSpec((blk, e.shape[1]), lambda ib, j: (j * nb + ib, 0)),
            pl.BlockSpec(ws.shape, lambda ib, j: (0, 0)),
            pl.BlockSpec(we.shape, lambda ib, j: (0, 0)),
            pl.BlockSpec((1, d2), lambda ib, j: (0, 0)),
            pl.BlockSpec((1, d2), lambda ib, j: (0, 0)),
            pl.BlockSpec((1, d2), lambda ib, j: (0, 0)),
            pl.BlockSpec((1, d2), lambda ib, j: (0, 0)),
        ],
        out_specs=[
            pl.BlockSpec((blk, d), lambda ib, j: (ib, 0)),
            pl.BlockSpec((1, d), lambda ib, j: (0, 0)),
            pl.BlockSpec((1, d), lambda ib, j: (0, 0)),
        ],
        out_shape=[
            jax.ShapeDtypeStruct((n, d), jnp.float32),
            jax.ShapeDtypeStruct((1, d), jnp.float32),
            jax.ShapeDtypeStruct((1, d), jnp.float32),
        ],
        scratch_shapes=[pltpu.VMEM((blk, d2), jnp.float32)],
    )(x, g, e, ws, we, s1, q1, g1, b1p)


def _conv_resid(x, ns, s2, q2, g2, b2, wn, b1, blk=2000):
    n, d = x.shape
    nb = n // blk
    cnt = float(n)
    with_proj = wn is not None

    def body(x_ref, ns_ref, s2_ref, q2_ref, g2_ref, b2_ref, *rest):
        mu = s2_ref[...] / cnt
        var = q2_ref[...] / cnt - mu * mu
        scale = g2_ref[...] * lax.rsqrt(var + _EPS)
        shift = b2_ref[...] - mu * scale
        xn = _softplus(x_ref[...] + ns_ref[...] * scale + shift)
        if with_proj:
            wn_ref, b1_ref, ox_ref, oy_ref = rest
            ox_ref[...] = xn
            oy_ref[...] = (
                jnp.dot(xn, wn_ref[...], preferred_element_type=jnp.float32,
                        precision=_PREC)
                + b1_ref[...]
            )
        else:
            (ox_ref,) = rest
            ox_ref[...] = xn

    in_specs = [
        pl.BlockSpec((blk, d), lambda i: (i, 0)),
        pl.BlockSpec((blk, d), lambda i: (i, 0)),
        pl.BlockSpec((1, d), lambda i: (0, 0)),
        pl.BlockSpec((1, d), lambda i: (0, 0)),
        pl.BlockSpec((1, d), lambda i: (0, 0)),
        pl.BlockSpec((1, d), lambda i: (0, 0)),
    ]
    args = [x, ns, s2, q2, g2, b2]
    if with_proj:
        d2 = wn.shape[1]
        in_specs += [
            pl.BlockSpec(wn.shape, lambda i: (0, 0)),
            pl.BlockSpec(b1.shape, lambda i: (0, 0)),
        ]
        args += [wn, b1]
        out_specs = [
            pl.BlockSpec((blk, d), lambda i: (i, 0)),
            pl.BlockSpec((blk, d2), lambda i: (i, 0)),
        ]
        out_shape = [
            jax.ShapeDtypeStruct((n, d), jnp.float32),
            jax.ShapeDtypeStruct((n, d2), jnp.float32),
        ]
    else:
        out_specs = pl.BlockSpec((blk, d), lambda i: (i, 0))
        out_shape = jax.ShapeDtypeStruct((n, d), jnp.float32)

    return pl.pallas_call(
        body,
        grid=(nb,),
        in_specs=in_specs,
        out_specs=out_specs,
        out_shape=out_shape,
    )(*args)


def _tail(x3, cfg_in, wfc, bfc, wcfg, bcfg, wfus, bfus, wout_t, bout):
    bsz, per, d = x3.shape

    def body(x3_ref, c_ref, wfc_ref, bfc_ref, wcfg_ref, bcfg_ref,
             wfus_ref, bfus_ref, wout_ref, bout_ref, o_ref):
        pooled = jnp.mean(x3_ref[...], axis=1)
        emb = _softplus(
            jnp.dot(pooled, wfc_ref[...], preferred_element_type=jnp.float32,
                    precision=_PREC)
            + bfc_ref[...]
        )
        cfg = jnp.maximum(
            jnp.dot(c_ref[...], wcfg_ref[...], preferred_element_type=jnp.float32,
                    precision=_PREC)
            + bcfg_ref[...],
            0.0,
        )
        fused = jnp.concatenate([emb, cfg], axis=1)
        fused = jnp.maximum(
            jnp.dot(fused, wfus_ref[...], preferred_element_type=jnp.float32,
                    precision=_PREC)
            + bfus_ref[...],
            0.0,
        )
        out = jnp.sum(fused * wout_ref[...], axis=1, keepdims=True) + bout_ref[...]
        o_ref[...] = out

    return pl.pallas_call(
        body,
        in_specs=[
            pl.BlockSpec(x3.shape, lambda: (0, 0, 0)),
            pl.BlockSpec(cfg_in.shape, lambda: (0, 0)),
            pl.BlockSpec(wfc.shape, lambda: (0, 0)),
            pl.BlockSpec(bfc.shape, lambda: (0, 0)),
            pl.BlockSpec(wcfg.shape, lambda: (0, 0)),
            pl.BlockSpec(bcfg.shape, lambda: (0, 0)),
            pl.BlockSpec(wfus.shape, lambda: (0, 0)),
            pl.BlockSpec(bfus.shape, lambda: (0, 0)),
            pl.BlockSpec(wout_t.shape, lambda: (0, 0)),
            pl.BlockSpec(bout.shape, lambda: (0, 0)),
        ],
        out_specs=pl.BlockSpec((bsz, 1), lambda: (0, 0)),
        out_shape=jax.ShapeDtypeStruct((bsz, 1), jnp.float32),
    )(x3, cfg_in, wfc, bfc, wcfg, bcfg, wfus, bfus, wout_t, bout)


def kernel(atom, nbr, idx, crys_idx, config_vector, mono_bg, W_emb, b_emb,
           conv_W, conv_b, bn1_g, bn1_b, bn2_g, bn2_b, W_fc, b_fc, W_cfg,
           b_cfg, W_prop, b_prop, W_fus, b_fus, W_out, b_out):
    n, _ = atom.shape
    m = idx.shape[1]
    d = W_emb.shape[1]
    nbrf = nbr.shape[2]
    nconv = conv_W.shape[0]
    blk = 5000

    wns = [conv_W[l][d:2 * d] for l in range(nconv)]
    wss = [conv_W[l][:d] for l in range(nconv)]
    wes = [conv_W[l][2 * d:] for l in range(nconv)]
    b1s = [conv_b[l].reshape(1, -1) for l in range(nconv)]

    x, y = _embed_proj(atom, W_emb, b_emb.reshape(1, -1), wns[0], b1s[0])

    idx_t = idx.T.reshape(-1)
    nbr_t = jnp.transpose(nbr, (1, 0, 2)).reshape(m * n, nbrf)

    for l in range(nconv):
        g = _sc_gather(y, idx_t)
        s1, q1 = _conv_stats(x, g, nbr_t, wss[l], wes[l], blk)
        ns, s2, q2 = _conv_apply(
            x, g, nbr_t, wss[l], wes[l], s1, q1,
            bn1_g[l].reshape(1, -1), bn1_b[l].reshape(1, -1), blk,
        )
        last = l == nconv - 1
        res = _conv_resid(
            x, ns, s2, q2, bn2_g[l].reshape(1, -1), bn2_b[l].reshape(1, -1),
            None if last else wns[l + 1], None if last else b1s[l + 1],
        )
        x = res if last else res[0]
        y = None if last else res[1]

    bsz = crys_idx.shape[0]
    x3 = x.reshape(bsz, n // bsz, d)
    return _tail(
        x3, config_vector, W_fc, b_fc.reshape(1, -1), W_cfg,
        b_cfg.reshape(1, -1), W_fus, b_fus.reshape(1, -1),
        W_out.T, b_out.reshape(1, 1),
    )
